# Initial kernel scaffold; baseline (speedup 1.0000x reference)
import functools

import jax
import jax.numpy as jnp
from jax import lax
from jax.experimental import pallas as pl
from jax.experimental.pallas import tpu as pltpu
from jax.experimental.pallas import tpu_sc as plsc

B, S, D = 4096, 200, 64
TAB_ROWS = S + 1
NC, NS = 2, 16
NW = NC * NS
BPW = B // NW
CB = 4
NCHUNK = BPW // CB
L = 16
DV = D // L


@functools.partial(
    pl.kernel,
    mesh=plsc.VectorSubcoreMesh(core_axis_name="c", subcore_axis_name="s"),
    out_type=jax.ShapeDtypeStruct((B * S * D,), jnp.float32),
    scratch_types=[
        pltpu.VMEM((CB * S * D,), jnp.float32),
        pltpu.VMEM((CB * S,), jnp.float32),
        pltpu.VMEM((TAB_ROWS * D,), jnp.float32),
    ],
)
def _pos_enc_sc(x_hbm, mask_hbm, tab_hbm, out_hbm, xbuf, mbuf, tbuf):
    wid = lax.axis_index("s") * NC + lax.axis_index("c")

    pltpu.sync_copy(tab_hbm, tbuf)

    def chunk_body(c, carry):
        base = (wid * BPW + c * CB) * S
        pltpu.sync_copy(x_hbm.at[pl.ds(base * D, CB * S * D)], xbuf)
        pltpu.sync_copy(mask_hbm.at[pl.ds(base, CB * S)], mbuf)

        def row_body(r, carry2):
            s_pos = r % S
            mvec = plsc.load_gather(mbuf, [jnp.full((L,), r, jnp.int32)])
            keep = 1.0 - mvec
            for d in range(DV):
                xo = r * D + d * L
                to = (s_pos + 1) * D + d * L
                xbuf[pl.ds(xo, L)] = (
                    xbuf[pl.ds(xo, L)] + tbuf[pl.ds(to, L)] * keep
                )
            return carry2

        lax.fori_loop(0, CB * S, row_body, 0)
        pltpu.sync_copy(xbuf, out_hbm.at[pl.ds(base * D, CB * S * D)])
        return carry

    lax.fori_loop(0, NCHUNK, chunk_body, 0)


def kernel(x, key_padding_mask, pos_table):
    x_flat = x.reshape(B * S * D)
    mask_f = key_padding_mask.astype(jnp.float32).reshape(B * S)
    tab_flat = pos_table.reshape(TAB_ROWS * D)
    out = _pos_enc_sc(x_flat, mask_f, tab_flat)
    return out.reshape(B, S, D)


# speedup vs baseline: 1.7211x; 1.7211x over previous
import functools

import jax
import jax.numpy as jnp
from jax import lax
from jax.experimental import pallas as pl
from jax.experimental.pallas import tpu as pltpu
from jax.experimental.pallas import tpu_sc as plsc

B, S, D = 4096, 200, 64
TAB_ROWS = S + 1
NC, NS = 2, 16
NW = NC * NS
BPW = B // NW
CB = 4
NCHUNK = BPW // CB
L = 16
DV = D // L


@functools.partial(
    pl.kernel,
    mesh=plsc.VectorSubcoreMesh(core_axis_name="c", subcore_axis_name="s"),
    out_type=jax.ShapeDtypeStruct((B * S * D,), jnp.float32),
    scratch_types=[
        pltpu.VMEM((CB * S * D,), jnp.float32),
        pltpu.VMEM((CB * S + L,), jnp.float32),
        pltpu.VMEM((TAB_ROWS * D,), jnp.float32),
    ],
)
def _pos_enc_sc(x_hbm, mask_hbm, tab_hbm, out_hbm, xbuf, mbuf, tbuf):
    wid = lax.axis_index("s") * NC + lax.axis_index("c")

    pltpu.sync_copy(tab_hbm, tbuf)

    def chunk_body(c, carry):
        base = (wid * BPW + c * CB) * S
        pltpu.sync_copy(x_hbm.at[pl.ds(base * D, CB * S * D)], xbuf)
        pltpu.sync_copy(mask_hbm.at[pl.ds(base, CB * S)], mbuf.at[pl.ds(0, CB * S)])

        def row_body(r, carry2):
            s_pos = r % S
            mvec = mbuf[pl.ds(r, L)]
            keep = 1.0 - jnp.broadcast_to(mvec[0], (L,))
            for d in range(DV):
                xo = r * D + d * L
                to = (s_pos + 1) * D + d * L
                xbuf[pl.ds(xo, L)] = (
                    xbuf[pl.ds(xo, L)] + tbuf[pl.ds(to, L)] * keep
                )
            return carry2

        lax.fori_loop(0, CB * S, row_body, 0)
        pltpu.sync_copy(xbuf, out_hbm.at[pl.ds(base * D, CB * S * D)])
        return carry

    lax.fori_loop(0, NCHUNK, chunk_body, 0)


def kernel(x, key_padding_mask, pos_table):
    x_flat = x.reshape(B * S * D)
    mask_f = key_padding_mask.astype(jnp.float32).reshape(B * S)
    tab_flat = pos_table.reshape(TAB_ROWS * D)
    out = _pos_enc_sc(x_flat, mask_f, tab_flat)
    return out.reshape(B, S, D)


# speedup vs baseline: 2.2046x; 1.2809x over previous
import functools

import jax
import jax.numpy as jnp
from jax import lax
from jax.experimental import pallas as pl
from jax.experimental.pallas import tpu as pltpu
from jax.experimental.pallas import tpu_sc as plsc

B, S, D = 4096, 200, 64
TAB_ROWS = S + 1
NC, NS = 2, 16
NW = NC * NS
BPW = B // NW
CB = 4
NCHUNK = BPW // CB
L = 16
DV = D // L


@functools.partial(
    pl.kernel,
    mesh=plsc.VectorSubcoreMesh(core_axis_name="c", subcore_axis_name="s"),
    out_type=jax.ShapeDtypeStruct((B * S * D,), jnp.float32),
    scratch_types=[
        pltpu.VMEM((CB * S * D,), jnp.float32),
        pltpu.VMEM((CB * S + L,), jnp.float32),
        pltpu.VMEM((TAB_ROWS * D,), jnp.float32),
    ],
)
def _pos_enc_sc(x_hbm, mask_hbm, tab_hbm, out_hbm, xbuf, mbuf, tbuf):
    wid = lax.axis_index("s") * NC + lax.axis_index("c")

    pltpu.sync_copy(tab_hbm, tbuf)

    def chunk_body(c, carry):
        base = (wid * BPW + c * CB) * S
        pltpu.sync_copy(x_hbm.at[pl.ds(base * D, CB * S * D)], xbuf)
        pltpu.sync_copy(mask_hbm.at[pl.ds(base, CB * S)], mbuf.at[pl.ds(0, CB * S)])

        def s_body(s, carry2):
            to = (s + 1) * D
            erow = [tbuf[pl.ds(to + d * L, L)] for d in range(DV)]
            for b in range(CB):
                r = b * S + s
                mvec = mbuf[pl.ds(r, L)]
                keep = 1.0 - jnp.broadcast_to(mvec[0], (L,))
                xo = r * D
                for d in range(DV):
                    xbuf[pl.ds(xo + d * L, L)] = (
                        xbuf[pl.ds(xo + d * L, L)] + erow[d] * keep
                    )
            return carry2

        lax.fori_loop(0, S, s_body, 0)
        pltpu.sync_copy(xbuf, out_hbm.at[pl.ds(base * D, CB * S * D)])
        return carry

    lax.fori_loop(0, NCHUNK, chunk_body, 0)


def kernel(x, key_padding_mask, pos_table):
    x_flat = x.reshape(B * S * D)
    mask_f = key_padding_mask.astype(jnp.float32).reshape(B * S)
    tab_flat = pos_table.reshape(TAB_ROWS * D)
    out = _pos_enc_sc(x_flat, mask_f, tab_flat)
    return out.reshape(B, S, D)


# speedup vs baseline: 2.4834x; 1.1265x over previous
import functools

import jax
import jax.numpy as jnp
from jax import lax
from jax.experimental import pallas as pl
from jax.experimental.pallas import tpu as pltpu
from jax.experimental.pallas import tpu_sc as plsc

B, S, D = 4096, 200, 64
TAB_ROWS = S + 1
NC, NS = 2, 16
NW = NC * NS
BPW = B // NW
CB = 4
NCHUNK = BPW // CB
L = 16
DV = D // L


@functools.partial(
    pl.kernel,
    mesh=plsc.VectorSubcoreMesh(core_axis_name="c", subcore_axis_name="s"),
    out_type=jax.ShapeDtypeStruct((B * S * D,), jnp.float32),
    scratch_types=[
        pltpu.VMEM((CB * S * D,), jnp.float32),
        pltpu.VMEM((CB * S + L,), jnp.float32),
        pltpu.VMEM((TAB_ROWS * D,), jnp.float32),
    ],
)
def _pos_enc_sc(x_hbm, mask_hbm, tab_hbm, out_hbm, xbuf, mbuf, tbuf):
    wid = lax.axis_index("s") * NC + lax.axis_index("c")

    pltpu.sync_copy(tab_hbm, tbuf)

    def chunk_body(c, carry):
        base = (wid * BPW + c * CB) * S
        pltpu.sync_copy(x_hbm.at[pl.ds(base * D, CB * S * D)], xbuf)
        pltpu.sync_copy(mask_hbm.at[pl.ds(base, CB * S)], mbuf.at[pl.ds(0, CB * S)])

        @plsc.parallel_loop(0, S, unroll=2)
        def s_body(s):
            to = (s + 1) * D
            erow = [tbuf[pl.ds(to + d * L, L)] for d in range(DV)]
            for b in range(CB):
                r = b * S + s
                mvec = mbuf[pl.ds(r, L)]
                keep = 1.0 - jnp.broadcast_to(mvec[0], (L,))
                xo = r * D
                for d in range(DV):
                    xbuf[pl.ds(xo + d * L, L)] = (
                        xbuf[pl.ds(xo + d * L, L)] + erow[d] * keep
                    )

        pltpu.sync_copy(xbuf, out_hbm.at[pl.ds(base * D, CB * S * D)])
        return carry

    lax.fori_loop(0, NCHUNK, chunk_body, 0)


def kernel(x, key_padding_mask, pos_table):
    x_flat = x.reshape(B * S * D)
    mask_f = key_padding_mask.astype(jnp.float32).reshape(B * S)
    tab_flat = pos_table.reshape(TAB_ROWS * D)
    out = _pos_enc_sc(x_flat, mask_f, tab_flat)
    return out.reshape(B, S, D)


# speedup vs baseline: 2.4885x; 1.0021x over previous
import functools

import jax
import jax.numpy as jnp
from jax import lax
from jax.experimental import pallas as pl
from jax.experimental.pallas import tpu as pltpu
from jax.experimental.pallas import tpu_sc as plsc

B, S, D = 4096, 200, 64
TAB_ROWS = S + 1
NC, NS = 2, 16
NW = NC * NS
BPW = B // NW
CB = 4
NCHUNK = BPW // CB
L = 16
DV = D // L


@functools.partial(
    pl.kernel,
    mesh=plsc.VectorSubcoreMesh(core_axis_name="c", subcore_axis_name="s"),
    out_type=jax.ShapeDtypeStruct((B * S * D,), jnp.float32),
    scratch_types=[
        pltpu.VMEM((CB * S * D,), jnp.float32),
        pltpu.VMEM((CB * S + L,), jnp.float32),
        pltpu.VMEM((TAB_ROWS * D,), jnp.float32),
    ],
)
def _pos_enc_sc(x_hbm, mask_hbm, tab_hbm, out_hbm, xbuf, mbuf, tbuf):
    wid = lax.axis_index("s") * NC + lax.axis_index("c")

    pltpu.sync_copy(tab_hbm, tbuf)

    def chunk_body(c, carry):
        base = (wid * BPW + c * CB) * S
        pltpu.sync_copy(x_hbm.at[pl.ds(base * D, CB * S * D)], xbuf)
        pltpu.sync_copy(mask_hbm.at[pl.ds(base, CB * S)], mbuf.at[pl.ds(0, CB * S)])

        @plsc.parallel_loop(0, S, unroll=4)
        def s_body(s):
            to = (s + 1) * D
            erow = [tbuf[pl.ds(to + d * L, L)] for d in range(DV)]
            for b in range(CB):
                r = b * S + s
                mvec = mbuf[pl.ds(r, L)]
                keep = 1.0 - jnp.broadcast_to(mvec[0], (L,))
                xo = r * D
                for d in range(DV):
                    xbuf[pl.ds(xo + d * L, L)] = (
                        xbuf[pl.ds(xo + d * L, L)] + erow[d] * keep
                    )

        pltpu.sync_copy(xbuf, out_hbm.at[pl.ds(base * D, CB * S * D)])
        return carry

    lax.fori_loop(0, NCHUNK, chunk_body, 0)


def kernel(x, key_padding_mask, pos_table):
    x_flat = x.reshape(B * S * D)
    mask_f = key_padding_mask.astype(jnp.float32).reshape(B * S)
    tab_flat = pos_table.reshape(TAB_ROWS * D)
    out = _pos_enc_sc(x_flat, mask_f, tab_flat)
    return out.reshape(B, S, D)


# speedup vs baseline: 3.0781x; 1.2369x over previous
import functools

import jax
import jax.numpy as jnp
from jax import lax
from jax.experimental import pallas as pl
from jax.experimental.pallas import tpu as pltpu
from jax.experimental.pallas import tpu_sc as plsc

B, S, D = 4096, 200, 64
TAB_ROWS = S + 1
NC, NS = 2, 16
NW = NC * NS
BPW = B // NW
CB = 4
NCHUNK = BPW // CB
L = 16
DV = D // L


@functools.partial(
    pl.kernel,
    mesh=plsc.VectorSubcoreMesh(core_axis_name="c", subcore_axis_name="s"),
    out_type=jax.ShapeDtypeStruct((B, S, D), jnp.float32),
    scratch_types=[
        pltpu.VMEM((CB, S, D), jnp.float32),
        pltpu.VMEM((CB * S + L,), jnp.float32),
        pltpu.VMEM((TAB_ROWS * D,), jnp.float32),
    ],
)
def _pos_enc_sc(x_hbm, mask_hbm, tab_hbm, out_hbm, xbuf, mbuf, tbuf):
    wid = lax.axis_index("s") * NC + lax.axis_index("c")

    pltpu.sync_copy(tab_hbm, tbuf)

    def chunk_body(c, carry):
        b0 = wid * BPW + c * CB
        pltpu.sync_copy(x_hbm.at[pl.ds(b0, CB)], xbuf)
        pltpu.sync_copy(mask_hbm.at[pl.ds(b0 * S, CB * S)], mbuf.at[pl.ds(0, CB * S)])

        @plsc.parallel_loop(0, S, unroll=4)
        def s_body(s):
            to = (s + 1) * D
            erow = [tbuf[pl.ds(to + d * L, L)] for d in range(DV)]
            for b in range(CB):
                mvec = mbuf[pl.ds(b * S + s, L)]
                keep = 1.0 - jnp.broadcast_to(mvec[0], (L,))
                for d in range(DV):
                    xbuf[b, s, pl.ds(d * L, L)] = (
                        xbuf[b, s, pl.ds(d * L, L)] + erow[d] * keep
                    )

        pltpu.sync_copy(xbuf, out_hbm.at[pl.ds(b0, CB)])
        return carry

    lax.fori_loop(0, NCHUNK, chunk_body, 0)


def kernel(x, key_padding_mask, pos_table):
    mask_f = key_padding_mask.astype(jnp.float32).reshape(B * S)
    tab_flat = pos_table.reshape(TAB_ROWS * D)
    return _pos_enc_sc(x, mask_f, tab_flat)


# speedup vs baseline: 11.8670x; 3.8553x over previous
import functools

import jax
import jax.numpy as jnp
from jax import lax
from jax.experimental import pallas as pl
from jax.experimental.pallas import tpu as pltpu
from jax.experimental.pallas import tpu_sc as plsc

B, S, D = 4096, 200, 64
TAB_ROWS = S + 1
NC, NS = 2, 16
NW = NC * NS
BW = B // NW
CS = 8
NCHUNK = S // CS
L = 16
NG = BW // L


@functools.partial(
    pl.kernel,
    mesh=plsc.VectorSubcoreMesh(core_axis_name="c", subcore_axis_name="s"),
    out_type=jax.ShapeDtypeStruct((S, D, B), jnp.float32),
    scratch_types=[
        pltpu.VMEM((CS, D, BW), jnp.float32),
        pltpu.VMEM((CS, BW), jnp.float32),
        pltpu.VMEM((TAB_ROWS * D,), jnp.float32),
    ],
)
def _pos_enc_sc(x_hbm, mask_hbm, tab_hbm, out_hbm, xbuf, mbuf, tbuf):
    wid = lax.axis_index("s") * NC + lax.axis_index("c")
    b0 = wid * BW

    pltpu.sync_copy(tab_hbm, tbuf)

    def chunk_body(c, carry):
        s0 = c * CS
        pltpu.sync_copy(x_hbm.at[pl.ds(s0, CS), :, pl.ds(b0, BW)], xbuf)
        pltpu.sync_copy(mask_hbm.at[pl.ds(s0, CS), pl.ds(b0, BW)], mbuf)

        @plsc.parallel_loop(0, CS, unroll=1)
        def s_body(sl):
            keep = [1.0 - mbuf[sl, pl.ds(g * L, L)] for g in range(NG)]
            row = (s0 + sl + 1) * D
            for dblk in range(D // L):
                tv = tbuf[pl.ds(row + dblk * L, L)]
                for j in range(L):
                    d = dblk * L + j
                    bval = jnp.broadcast_to(tv[j], (L,))
                    for g in range(NG):
                        xbuf[sl, d, pl.ds(g * L, L)] = (
                            xbuf[sl, d, pl.ds(g * L, L)] + bval * keep[g]
                        )

        pltpu.sync_copy(xbuf, out_hbm.at[pl.ds(s0, CS), :, pl.ds(b0, BW)])
        return carry

    lax.fori_loop(0, NCHUNK, chunk_body, 0)


def kernel(x, key_padding_mask, pos_table):
    x_t = jnp.transpose(x, (1, 2, 0))
    mask_f = key_padding_mask.T.astype(jnp.float32)
    tab_flat = pos_table.reshape(TAB_ROWS * D)
    out_t = _pos_enc_sc(x_t, mask_f, tab_flat)
    return jnp.transpose(out_t, (2, 0, 1))


# speedup vs baseline: 15.6876x; 1.3219x over previous
import functools

import jax
import jax.numpy as jnp
from jax import lax
from jax.experimental import pallas as pl
from jax.experimental.pallas import tpu as pltpu
from jax.experimental.pallas import tpu_sc as plsc

B, S, D = 4096, 200, 64
TAB_ROWS = S + 1
NC, NS = 2, 16
NW = NC * NS
BW = B // NW
CS = 2
NCHUNK = S // CS
L = 16
NG = BW // L


@functools.partial(
    pl.kernel,
    mesh=plsc.VectorSubcoreMesh(core_axis_name="c", subcore_axis_name="s"),
    out_type=jax.ShapeDtypeStruct((S, D, B), jnp.float32),
    scratch_types=[
        pltpu.VMEM((CS, D, BW), jnp.float32),
        pltpu.VMEM((CS, D, BW), jnp.float32),
        pltpu.VMEM((CS, D, BW), jnp.float32),
        pltpu.VMEM((CS, D, BW), jnp.float32),
        pltpu.VMEM((CS, BW), jnp.float32),
        pltpu.VMEM((CS, BW), jnp.float32),
        pltpu.VMEM((TAB_ROWS * D,), jnp.float32),
        pltpu.SemaphoreType.DMA,
        pltpu.SemaphoreType.DMA,
        pltpu.SemaphoreType.DMA,
        pltpu.SemaphoreType.DMA,
    ],
)
def _pos_enc_sc(x_hbm, mask_hbm, tab_hbm, out_hbm,
                xin0, xin1, xout0, xout1, mb0, mb1, tbuf,
                isem0, isem1, osem0, osem1):
    wid = lax.axis_index("s") * NC + lax.axis_index("c")
    b0 = wid * BW
    xin = (xin0, xin1)
    xout = (xout0, xout1)
    mb = (mb0, mb1)
    isem = (isem0, isem1)
    osem = (osem0, osem1)

    pltpu.sync_copy(tab_hbm, tbuf)

    def start_in(c, i):
        s0 = c * CS
        pltpu.async_copy(x_hbm.at[pl.ds(s0, CS), :, pl.ds(b0, BW)],
                         xin[i], isem[i])
        pltpu.async_copy(mask_hbm.at[pl.ds(s0, CS), pl.ds(b0, BW)],
                         mb[i], isem[i])

    def wait_in(i):
        pltpu.make_async_copy(x_hbm.at[pl.ds(0, CS), :, pl.ds(b0, BW)],
                              xin[i], isem[i]).wait()
        pltpu.make_async_copy(mask_hbm.at[pl.ds(0, CS), pl.ds(b0, BW)],
                              mb[i], isem[i]).wait()

    def start_out(c, i):
        s0 = c * CS
        pltpu.async_copy(xout[i], out_hbm.at[pl.ds(s0, CS), :, pl.ds(b0, BW)],
                         osem[i])

    def wait_out(i):
        pltpu.make_async_copy(xout[i], out_hbm.at[pl.ds(0, CS), :, pl.ds(b0, BW)],
                              osem[i]).wait()

    def compute(c, i):
        s0 = c * CS
        src, dst, msk = xin[i], xout[i], mb[i]

        @plsc.parallel_loop(0, CS, unroll=1)
        def s_body(sl):
            keep = [1.0 - msk[sl, pl.ds(g * L, L)] for g in range(NG)]
            row = (s0 + sl + 1) * D
            for dblk in range(D // L):
                tv = tbuf[pl.ds(row + dblk * L, L)]
                for j in range(L):
                    d = dblk * L + j
                    bval = jnp.broadcast_to(tv[j], (L,))
                    for g in range(NG):
                        dst[sl, d, pl.ds(g * L, L)] = (
                            src[sl, d, pl.ds(g * L, L)] + bval * keep[g]
                        )

    start_in(0, 0)

    def pair_body(g, carry):
        for i in range(2):
            c = 2 * g + i
            wait_in(i)

            @pl.when(g >= 1)
            def _():
                wait_out(i)

            @pl.when(c + 1 < NCHUNK)
            def _():
                start_in(c + 1, (i + 1) % 2)

            compute(c, i)
            start_out(c, i)
        return carry

    lax.fori_loop(0, NCHUNK // 2, pair_body, 0)
    wait_out(0)
    wait_out(1)


def kernel(x, key_padding_mask, pos_table):
    x_t = jnp.transpose(x, (1, 2, 0))
    mask_f = key_padding_mask.T.astype(jnp.float32)
    tab_flat = pos_table.reshape(TAB_ROWS * D)
    out_t = _pos_enc_sc(x_t, mask_f, tab_flat)
    return jnp.transpose(out_t, (2, 0, 1))
